# baseline (speedup 1.0000x reference)
import functools

import jax
import jax.numpy as jnp
from jax import lax
from jax.experimental import pallas as pl
from jax.experimental.pallas import tpu as pltpu
from jax.experimental.pallas import tpu_sc as plsc

_D = 64
_NFS = 16
_NW = 32
_CHUNK = 128

_DEBUG_W_GATHER = True
_DEBUG_B_GATHER = True
_DEBUG_NO_COMPUTE = False
_LAYOUT_PASSES = False


@functools.lru_cache(maxsize=None)
def _build(B, n_rows):
    n_per_w = B // _NW
    n_dma = n_per_w // _CHUNK
    n_groups = n_per_w // 16

    mesh = plsc.VectorSubcoreMesh(core_axis_name="c", subcore_axis_name="s")

    @functools.partial(
        pl.kernel,
        mesh=mesh,
        out_type=jax.ShapeDtypeStruct((B,), jnp.float32),
        compiler_params=pltpu.CompilerParams(
            needs_layout_passes=_LAYOUT_PASSES, use_tc_tiling_on_sc=False),
        scratch_types=[
            pltpu.VMEM((n_per_w, _D), jnp.float32),
            pltpu.VMEM((n_per_w, _D), jnp.float32),
            pltpu.VMEM((n_per_w,), jnp.float32),
            pltpu.VMEM((n_per_w,), jnp.int32),
            pltpu.VMEM((n_per_w,), jnp.int32),
            pltpu.VMEM((n_per_w,), jnp.int32),
            [pltpu.VMEM((_CHUNK,), jnp.int32)] * 4,
            pltpu.VMEM((_CHUNK, _D), jnp.float32),
            pltpu.VMEM((n_per_w,), jnp.float32),
            pltpu.VMEM((256,), jnp.float32),
            pltpu.SemaphoreType.DMA,
            pltpu.SemaphoreType.DMA,
            pltpu.SemaphoreType.DMA,
            pltpu.SemaphoreType.DMA,
        ],
    )
    def k(feat_hbm, stn_hbm, fct_hbm, stp_hbm, wt_hbm, bias_hbm, out_hbm,
          feat_v, rows_v, bias_v, stn_v, fct_v, stp_v, idx_c, rows0, out_v,
          stage, sem_in, sem_id, sem_w, sem_b):
        wid = lax.axis_index("s") * 2 + lax.axis_index("c")
        base = wid * n_per_w

        cp_f = pltpu.async_copy(feat_hbm.at[pl.ds(base, n_per_w)], feat_v, sem_in)
        cp_s = pltpu.async_copy(stn_hbm.at[pl.ds(base, n_per_w)], stn_v, sem_id)
        cp_c = pltpu.async_copy(fct_hbm.at[pl.ds(base, n_per_w)], fct_v, sem_id)
        cp_p = pltpu.async_copy(stp_hbm.at[pl.ds(base, n_per_w)], stp_v, sem_id)
        cp_s.wait()
        cp_c.wait()
        cp_p.wait()

        for c in range(n_per_w // 16):
            o = c * 16
            st = stn_v[pl.ds(o, 16)]
            fo = fct_v[pl.ds(o, 16)]
            sp = stp_v[pl.ds(o, 16)]
            idx = st * _NFS + (fo & 1) * 8 + (sp & 7)
            kk = o // _CHUNK
            idx_c[kk][pl.ds(o - kk * _CHUNK, 16)] = idx

        descs = []
        for kk in range(n_dma):
            r0 = kk * _CHUNK
            if _DEBUG_W_GATHER:
                descs.append(pltpu.async_copy(
                    wt_hbm.at[idx_c[kk]], rows_v.at[pl.ds(r0, _CHUNK)],
                    sem_w))
            if _DEBUG_B_GATHER:
                descs.append(pltpu.async_copy(
                    bias_hbm.at[idx_c[kk]], bias_v.at[pl.ds(r0, _CHUNK)],
                    sem_b))
        cp_f.wait()
        for d in descs:
            d.wait()

        if _DEBUG_NO_COMPUTE:
            for c in range(n_groups):
                o = c * 16
                out_v[pl.ds(o, 16)] = (bias_v[pl.ds(o, 16)]
                                       + rows0[c, pl.ds(0, 16)]
                                       + feat_v[c, pl.ds(0, 16)])
            pltpu.sync_copy(out_v, out_hbm.at[pl.ds(base, n_per_w)])
            return

        lane16 = lax.iota(jnp.int32, 16) * 16

        def group(g, carry):
            base_e = g * 16
            for q in range(16):
                e = base_e + q
                acc = feat_v[e, pl.ds(0, 16)] * rows_v[e, pl.ds(0, 16)]
                for t in range(1, _D // 16):
                    acc = acc + feat_v[e, pl.ds(t * 16, 16)] * rows_v[e, pl.ds(t * 16, 16)]
                stage[pl.ds(q * 16, 16)] = acc
            res = bias_v[pl.ds(base_e, 16)]
            for t in range(16):
                res = res + plsc.load_gather(stage, [lane16 + t])
            out_v[pl.ds(base_e, 16)] = res
            return carry

        lax.fori_loop(0, n_groups, group, 0)
        pltpu.sync_copy(out_v, out_hbm.at[pl.ds(base, n_per_w)])

    return k


def kernel(features, station_id, forecast_id, step_id, weights, biases):
    B = features.shape[0]
    n_rows = weights.shape[0] * weights.shape[1] * weights.shape[2]
    wt = weights.reshape(n_rows, _D)
    bs = biases.reshape(n_rows)
    k = _build(B, n_rows)
    return k(features,
             station_id.astype(jnp.int32),
             forecast_id.astype(jnp.int32),
             step_id.astype(jnp.int32),
             wt, bs)
